# baseline (speedup 1.0000x reference)
import jax
import jax.numpy as jnp
from jax import lax
from jax.experimental import pallas as pl
from jax.experimental.pallas import tpu as pltpu, tpu_sc as plsc

NUM_WORKERS = 32
L = 16
V = 1000
D = 32
DP = D + 1
C = 20

B, W = 4096, 50
N_WORDS = B * W
WORDS_PER_TILE = N_WORDS // NUM_WORKERS
CHUNK_WORDS = 640
NUM_CHUNKS = WORDS_PER_TILE // CHUNK_WORDS
CHUNK_IDS = CHUNK_WORDS * C
GROUPS = CHUNK_WORDS // L


def _sc_body(ids_hbm, table_hbm, out_hbm, table_v, ids_v, out_v):
    wid = lax.axis_index("s") * 2 + lax.axis_index("c")
    word_base = wid * WORDS_PER_TILE

    pltpu.sync_copy(table_hbm, table_v)
    zeros = jnp.zeros((L,), jnp.float32)
    table_v[0, pl.ds(0, L)] = zeros
    table_v[0, pl.ds(L, L)] = zeros

    lanes = lax.iota(jnp.int32, L)

    def chunk_body(g, carry):
        chunk_word0 = word_base + g * CHUNK_WORDS
        pltpu.sync_copy(ids_hbm.at[pl.ds(chunk_word0 * C, CHUNK_IDS)], ids_v)

        def group_body(gi, carry2):
            w0 = gi * L
            id_base = (w0 + lanes) * C
            idvs = [plsc.load_gather(ids_v, [id_base + c]) for c in range(C)]
            words = w0 + lanes

            def col_body(d, carry3):
                colv = jnp.full((L,), d, jnp.int32)
                vals = [plsc.load_gather(table_v, [idvs[c], colv]) for c in range(C)]
                while len(vals) > 1:
                    nxt = [vals[i] + vals[i + 1] for i in range(0, len(vals) - 1, 2)]
                    if len(vals) % 2:
                        nxt.append(vals[-1])
                    vals = nxt
                plsc.store_scatter(out_v, [words, colv], vals[0])
                return carry3

            lax.fori_loop(0, D, col_body, 0)
            return carry2

        lax.fori_loop(0, GROUPS, group_body, 0)
        pltpu.sync_copy(
            out_v.at[:, pl.ds(0, D)],
            out_hbm.at[pl.ds(chunk_word0, CHUNK_WORDS), :],
        )
        return carry

    lax.fori_loop(0, NUM_CHUNKS, chunk_body, 0)


@jax.jit
def kernel(token_ids, table):
    ids_flat = token_ids.astype(jnp.int32).reshape(-1)
    table_p = jnp.pad(table, ((0, 0), (0, DP - D)))
    sc_call = pl.kernel(
        _sc_body,
        out_type=jax.ShapeDtypeStruct((N_WORDS, D), jnp.float32),
        mesh=plsc.VectorSubcoreMesh(core_axis_name="c", subcore_axis_name="s"),
        compiler_params=pltpu.CompilerParams(
            needs_layout_passes=False, use_tc_tiling_on_sc=False
        ),
        scratch_types=[
            pltpu.VMEM((V, DP), jnp.float32),
            pltpu.VMEM((CHUNK_IDS,), jnp.int32),
            pltpu.VMEM((CHUNK_WORDS, DP), jnp.float32),
        ],
    )
    out = sc_call(ids_flat, table_p)
    return out.reshape(B, W, D)


# speedup vs baseline: 2.2287x; 2.2287x over previous
import jax
import jax.numpy as jnp
from jax import lax
from jax.experimental import pallas as pl
from jax.experimental.pallas import tpu as pltpu, tpu_sc as plsc

NUM_WORKERS = 32
L = 16
V = 1000
D = 32
C = 20

B, W = 4096, 50
N_WORDS = B * W
WORDS_PER_TILE = N_WORDS // NUM_WORKERS
CHUNK_WORDS = 640
NUM_CHUNKS = WORDS_PER_TILE // CHUNK_WORDS
CHUNK_IDS = CHUNK_WORDS * C


def _tree_sum(vals):
    while len(vals) > 1:
        nxt = [vals[i] + vals[i + 1] for i in range(0, len(vals) - 1, 2)]
        if len(vals) % 2:
            nxt.append(vals[-1])
        vals = nxt
    return vals[0]


def _sc_body(ids_hbm, table_hbm, out_hbm, table_v, ids_v, out_v):
    wid = lax.axis_index("s") * 2 + lax.axis_index("c")
    word_base = wid * WORDS_PER_TILE

    pltpu.sync_copy(table_hbm, table_v)
    zeros = jnp.zeros((L,), jnp.float32)
    table_v[0, pl.ds(0, L)] = zeros
    table_v[0, pl.ds(L, L)] = zeros

    def chunk_body(g, carry):
        chunk_word0 = word_base + g * CHUNK_WORDS
        pltpu.sync_copy(
            ids_hbm.at[pl.ds(chunk_word0 * C, CHUNK_IDS)],
            ids_v.at[pl.ds(0, CHUNK_IDS)],
        )

        def word_body(w, carry2):
            base = w * C
            idv0 = ids_v[pl.ds(base, L)]
            idv1 = ids_v[pl.ds(base + L, L)]
            lo = []
            hi = []
            for c in range(C):
                idx = idv0[c] if c < L else idv1[c - L]
                lo.append(table_v[idx, pl.ds(0, L)])
                hi.append(table_v[idx, pl.ds(L, L)])
            out_v[pl.ds(w * D, L)] = _tree_sum(lo)
            out_v[pl.ds(w * D + L, L)] = _tree_sum(hi)
            return carry2

        lax.fori_loop(0, CHUNK_WORDS, word_body, 0)
        pltpu.sync_copy(
            out_v, out_hbm.at[pl.ds(chunk_word0 * D, CHUNK_WORDS * D)]
        )
        return carry

    lax.fori_loop(0, NUM_CHUNKS, chunk_body, 0)


@jax.jit
def kernel(token_ids, table):
    ids_flat = token_ids.astype(jnp.int32).reshape(-1)
    sc_call = pl.kernel(
        _sc_body,
        out_type=jax.ShapeDtypeStruct((N_WORDS * D,), jnp.float32),
        mesh=plsc.VectorSubcoreMesh(core_axis_name="c", subcore_axis_name="s"),
        compiler_params=pltpu.CompilerParams(
            needs_layout_passes=False, use_tc_tiling_on_sc=False
        ),
        scratch_types=[
            pltpu.VMEM((V, D), jnp.float32),
            pltpu.VMEM((CHUNK_IDS + L,), jnp.int32),
            pltpu.VMEM((CHUNK_WORDS * D,), jnp.float32),
        ],
    )
    out = sc_call(ids_flat, table)
    return out.reshape(B, W, D)


# speedup vs baseline: 2.9224x; 1.3112x over previous
import jax
import jax.numpy as jnp
from jax import lax
from jax.experimental import pallas as pl
from jax.experimental.pallas import tpu as pltpu, tpu_sc as plsc

NUM_WORKERS = 32
L = 16
V = 1000
D = 32
C = 20

B, W = 4096, 50
N_WORDS = B * W
WORDS_PER_TILE = N_WORDS // NUM_WORKERS
CHUNK_WORDS = 640
NUM_CHUNKS = WORDS_PER_TILE // CHUNK_WORDS
CHUNK_IDS = CHUNK_WORDS * C


def _tree_sum(vals):
    while len(vals) > 1:
        nxt = [vals[i] + vals[i + 1] for i in range(0, len(vals) - 1, 2)]
        if len(vals) % 2:
            nxt.append(vals[-1])
        vals = nxt
    return vals[0]


def _sc_body(ids_hbm, table_hbm, out_hbm, table_v, ids_v, out_v,
             si0, si1, so0, so1):
    wid = lax.axis_index("s") * 2 + lax.axis_index("c")
    word_base = wid * WORDS_PER_TILE
    ids_sems = [si0, si1]
    out_sems = [so0, so1]

    def start_ids(g):
        cw0 = word_base + g * CHUNK_WORDS
        return pltpu.async_copy(
            ids_hbm.at[pl.ds(cw0 * C, CHUNK_IDS)],
            ids_v.at[g % 2, pl.ds(0, CHUNK_IDS)],
            ids_sems[g % 2],
        )

    in_descs = {0: start_ids(0)}
    out_descs = {}

    pltpu.sync_copy(table_hbm, table_v)
    zeros = jnp.zeros((L,), jnp.float32)
    table_v[0, pl.ds(0, L)] = zeros
    table_v[0, pl.ds(L, L)] = zeros

    for g in range(NUM_CHUNKS):
        if g + 1 < NUM_CHUNKS:
            in_descs[g + 1] = start_ids(g + 1)
        in_descs[g].wait()
        if g >= 2:
            out_descs[g - 2].wait()
        gb = g % 2

        @plsc.parallel_loop(0, CHUNK_WORDS, step=1, unroll=2)
        def word_body(w):
            base = w * C
            idv0 = ids_v[gb, pl.ds(base, L)]
            idv1 = ids_v[gb, pl.ds(base + L, L)]
            lo = []
            hi = []
            for c in range(C):
                idx = idv0[c] if c < L else idv1[c - L]
                lo.append(table_v[idx, pl.ds(0, L)])
                hi.append(table_v[idx, pl.ds(L, L)])
            out_v[gb, pl.ds(w * D, L)] = _tree_sum(lo)
            out_v[gb, pl.ds(w * D + L, L)] = _tree_sum(hi)

        cw0 = word_base + g * CHUNK_WORDS
        out_descs[g] = pltpu.async_copy(
            out_v.at[gb],
            out_hbm.at[pl.ds(cw0 * D, CHUNK_WORDS * D)],
            out_sems[gb],
        )

    out_descs[NUM_CHUNKS - 2].wait()
    out_descs[NUM_CHUNKS - 1].wait()


@jax.jit
def kernel(token_ids, table):
    ids_flat = token_ids.astype(jnp.int32).reshape(-1)
    sc_call = pl.kernel(
        _sc_body,
        out_type=jax.ShapeDtypeStruct((N_WORDS * D,), jnp.float32),
        mesh=plsc.VectorSubcoreMesh(core_axis_name="c", subcore_axis_name="s"),
        compiler_params=pltpu.CompilerParams(
            needs_layout_passes=False, use_tc_tiling_on_sc=False
        ),
        scratch_types=[
            pltpu.VMEM((V, D), jnp.float32),
            pltpu.VMEM((2, CHUNK_IDS + L), jnp.int32),
            pltpu.VMEM((2, CHUNK_WORDS * D), jnp.float32),
            pltpu.SemaphoreType.DMA,
            pltpu.SemaphoreType.DMA,
            pltpu.SemaphoreType.DMA,
            pltpu.SemaphoreType.DMA,
        ],
    )
    out = sc_call(ids_flat, table)
    return out.reshape(B, W, D)


# speedup vs baseline: 3.0528x; 1.0446x over previous
import jax
import jax.numpy as jnp
from jax import lax
from jax.experimental import pallas as pl
from jax.experimental.pallas import tpu as pltpu, tpu_sc as plsc

NUM_WORKERS = 32
L = 16
V = 1000
D = 32
C = 20

B, W = 4096, 50
N_WORDS = B * W
WORDS_PER_TILE = N_WORDS // NUM_WORKERS
CHUNK_WORDS = 640
NUM_CHUNKS = WORDS_PER_TILE // CHUNK_WORDS
CHUNK_IDS = CHUNK_WORDS * C


def _tree_sum(vals):
    while len(vals) > 1:
        nxt = [vals[i] + vals[i + 1] for i in range(0, len(vals) - 1, 2)]
        if len(vals) % 2:
            nxt.append(vals[-1])
        vals = nxt
    return vals[0]


def _sc_body(ids_hbm, table_hbm, out_hbm, table_v, ids_v, out_v,
             si0, si1, so0, so1):
    wid = lax.axis_index("s") * 2 + lax.axis_index("c")
    word_base = wid * WORDS_PER_TILE
    ids_sems = [si0, si1]
    out_sems = [so0, so1]

    def start_ids(g):
        cw0 = word_base + g * CHUNK_WORDS
        return pltpu.async_copy(
            ids_hbm.at[pl.ds(cw0 * C, CHUNK_IDS)],
            ids_v.at[g % 2, pl.ds(0, CHUNK_IDS)],
            ids_sems[g % 2],
        )

    in_descs = {0: start_ids(0)}
    out_descs = {}

    pltpu.sync_copy(table_hbm, table_v)
    table_v[0, pl.ds(0, L)] = jnp.zeros((L,), jnp.int32)

    iota2 = lax.iota(jnp.int32, L) * 2

    for g in range(NUM_CHUNKS):
        if g + 1 < NUM_CHUNKS:
            in_descs[g + 1] = start_ids(g + 1)
        in_descs[g].wait()
        if g >= 2:
            out_descs[g - 2].wait()
        gb = g % 2

        @plsc.parallel_loop(0, CHUNK_WORDS, step=1, unroll=2)
        def word_body(w):
            base = w * C
            idv0 = ids_v[gb, pl.ds(base, L)]
            idv1 = ids_v[gb, pl.ds(base + L, L)]
            even = []
            odd = []
            for c in range(C):
                idx = idv0[c] if c < L else idv1[c - L]
                packed = table_v[idx, pl.ds(0, L)]
                pb = plsc.bitcast(packed, jnp.bfloat16)
                e, o = plsc.unpack(pb, format=plsc.PackFormat.INTERLEAVED)
                even.append(e)
                odd.append(o)
            obase = w * D + iota2
            plsc.store_scatter(out_v, [jnp.full((L,), gb, jnp.int32), obase],
                               _tree_sum(even))
            plsc.store_scatter(out_v, [jnp.full((L,), gb, jnp.int32), obase + 1],
                               _tree_sum(odd))

        cw0 = word_base + g * CHUNK_WORDS
        out_descs[g] = pltpu.async_copy(
            out_v.at[gb],
            out_hbm.at[pl.ds(cw0 * D, CHUNK_WORDS * D)],
            out_sems[gb],
        )

    out_descs[NUM_CHUNKS - 2].wait()
    out_descs[NUM_CHUNKS - 1].wait()


@jax.jit
def kernel(token_ids, table):
    ids_flat = token_ids.astype(jnp.int32).reshape(-1)
    tb = table.astype(jnp.bfloat16).reshape(V, L, 2)
    ti = lax.bitcast_convert_type(tb, jnp.int32)
    sc_call = pl.kernel(
        _sc_body,
        out_type=jax.ShapeDtypeStruct((N_WORDS * D,), jnp.float32),
        mesh=plsc.VectorSubcoreMesh(core_axis_name="c", subcore_axis_name="s"),
        compiler_params=pltpu.CompilerParams(
            needs_layout_passes=False, use_tc_tiling_on_sc=False
        ),
        scratch_types=[
            pltpu.VMEM((V, L), jnp.int32),
            pltpu.VMEM((2, CHUNK_IDS + L), jnp.int32),
            pltpu.VMEM((2, CHUNK_WORDS * D), jnp.float32),
            pltpu.SemaphoreType.DMA,
            pltpu.SemaphoreType.DMA,
            pltpu.SemaphoreType.DMA,
            pltpu.SemaphoreType.DMA,
        ],
    )
    out = sc_call(ids_flat, ti)
    return out.reshape(B, W, D)


# speedup vs baseline: 3.5508x; 1.1631x over previous
import jax
import jax.numpy as jnp
from jax import lax
from jax.experimental import pallas as pl
from jax.experimental.pallas import tpu as pltpu, tpu_sc as plsc

NUM_WORKERS = 32
L = 16
V = 1000
D = 32
C = 20

B, W = 4096, 50
N_WORDS = B * W
WORDS_PER_TILE = N_WORDS // NUM_WORKERS
CHUNK_WORDS = 640
NUM_CHUNKS = WORDS_PER_TILE // CHUNK_WORDS
CHUNK_IDS = CHUNK_WORDS * C


def _tree_sum(vals):
    while len(vals) > 1:
        nxt = [vals[i] + vals[i + 1] for i in range(0, len(vals) - 1, 2)]
        if len(vals) % 2:
            nxt.append(vals[-1])
        vals = nxt
    return vals[0]


def _sc_body(ids_hbm, table_hbm, out_hbm, table_v, ids_v, out_v,
             si0, si1, so0, so1):
    wid = lax.axis_index("s") * 2 + lax.axis_index("c")
    word_base = wid * WORDS_PER_TILE
    ids_sems = [si0, si1]
    out_sems = [so0, so1]

    def start_ids(g):
        cw0 = word_base + g * CHUNK_WORDS
        return pltpu.async_copy(
            ids_hbm.at[pl.ds(cw0 * C, CHUNK_IDS)],
            ids_v.at[g % 2, pl.ds(0, CHUNK_IDS)],
            ids_sems[g % 2],
        )

    in_descs = {0: start_ids(0)}
    out_descs = {}

    pltpu.sync_copy(table_hbm, table_v)
    table_v[0, pl.ds(0, L)] = jnp.zeros((L,), jnp.int32)

    for g in range(NUM_CHUNKS):
        if g + 1 < NUM_CHUNKS:
            in_descs[g + 1] = start_ids(g + 1)
        in_descs[g].wait()
        if g >= 2:
            out_descs[g - 2].wait()
        gb = g % 2

        @plsc.parallel_loop(0, CHUNK_WORDS, step=1, unroll=2)
        def word_body(w):
            base = w * C
            idv0 = ids_v[gb, pl.ds(base, L)]
            idv1 = ids_v[gb, pl.ds(base + L, L)]
            rows = []
            for c in range(C):
                idx = idv0[c] if c < L else idv1[c - L]
                packed = table_v[idx, pl.ds(0, L)]
                rows.append(plsc.bitcast(packed, jnp.bfloat16))
            s = _tree_sum(rows)
            lo, hi = plsc.unpack(s, format=plsc.PackFormat.INTERLEAVED)
            out_v[gb, pl.ds(w * D, L)] = lo
            out_v[gb, pl.ds(w * D + L, L)] = hi

        cw0 = word_base + g * CHUNK_WORDS
        out_descs[g] = pltpu.async_copy(
            out_v.at[gb],
            out_hbm.at[pl.ds(cw0 * D, CHUNK_WORDS * D)],
            out_sems[gb],
        )

    out_descs[NUM_CHUNKS - 2].wait()
    out_descs[NUM_CHUNKS - 1].wait()


@jax.jit
def kernel(token_ids, table):
    ids_flat = token_ids.astype(jnp.int32).reshape(-1)
    tb = table.astype(jnp.bfloat16)
    pairs = jnp.stack([tb[:, :L], tb[:, L:]], axis=-1)
    ti = lax.bitcast_convert_type(pairs, jnp.int32)
    sc_call = pl.kernel(
        _sc_body,
        out_type=jax.ShapeDtypeStruct((N_WORDS * D,), jnp.float32),
        mesh=plsc.VectorSubcoreMesh(core_axis_name="c", subcore_axis_name="s"),
        compiler_params=pltpu.CompilerParams(
            needs_layout_passes=False, use_tc_tiling_on_sc=False
        ),
        scratch_types=[
            pltpu.VMEM((V, L), jnp.int32),
            pltpu.VMEM((2, CHUNK_IDS + L), jnp.int32),
            pltpu.VMEM((2, CHUNK_WORDS * D), jnp.float32),
            pltpu.SemaphoreType.DMA,
            pltpu.SemaphoreType.DMA,
            pltpu.SemaphoreType.DMA,
            pltpu.SemaphoreType.DMA,
        ],
    )
    out = sc_call(ids_flat, ti)
    return out.reshape(B, W, D)
